# baseline (speedup 1.0000x reference)
import functools

import jax
import jax.numpy as jnp
from jax import lax
from jax.experimental import pallas as pl
from jax.experimental.pallas import tpu as pltpu
from jax.experimental.pallas import tpu_sc as plsc

N = 10000
F = 128
E = 320000
NC = 2
NS = 16
NW = NC * NS
EPW = E // NW
B = 40
NB = EPW // B
RPT = N // NS
RB = 1000

_mesh = plsc.VectorSubcoreMesh(core_axis_name="c", subcore_axis_name="s")
_sc_params = pltpu.CompilerParams(needs_layout_passes=False,
                                  use_tc_tiling_on_sc=False)


@functools.partial(
    pl.kernel,
    mesh=_mesh,
    compiler_params=_sc_params,
    out_type=jax.ShapeDtypeStruct((NW * N,), jnp.float32),
    scratch_types=[
        pltpu.VMEM((EPW,), jnp.int32),
        pltpu.VMEM((N,), jnp.float32),
        pltpu.VMEM((N,), jnp.float32),
        pltpu.VMEM((N,), jnp.float32),
        pltpu.VMEM((N,), jnp.float32),
    ],
)
def _sc_degree(dst_hbm, out_hbm, dst_v, acc_v, acc_v1, acc_v2, acc_v3):
    wid = lax.axis_index("s") * NC + lax.axis_index("c")
    accs = (acc_v, acc_v1, acc_v2, acc_v3)
    pltpu.sync_copy(dst_hbm.at[pl.ds(wid * EPW, EPW)], dst_v)
    zero = jnp.zeros((16,), jnp.float32)
    one = jnp.ones((16,), jnp.float32)

    def zbody(i, c):
        for a in accs:
            a[pl.ds(i * 16, 16)] = zero
        return c

    lax.fori_loop(0, N // 16, zbody, 0)

    def body(i, c):
        for u in range(4):
            ids = dst_v[pl.ds((i * 4 + u) * 16, 16)]
            plsc.addupdate_scatter(accs[u], [ids], one)
        return c

    lax.fori_loop(0, EPW // 64, body, 0)

    def mbody(i, c):
        sl = pl.ds(i * 16, 16)
        acc_v[sl] = ((acc_v[sl] + acc_v1[sl]) + (acc_v2[sl] + acc_v3[sl]))
        return c

    lax.fori_loop(0, N // 16, mbody, 0)
    pltpu.sync_copy(acc_v, out_hbm.at[pl.ds(wid * N, N)])


@functools.partial(
    pl.kernel,
    mesh=_mesh,
    compiler_params=_sc_params,
    out_type=jax.ShapeDtypeStruct((NC * N, F), jnp.float32),
    scratch_types=[
        pltpu.VMEM((EPW,), jnp.int32),
        pltpu.VMEM((EPW,), jnp.int32),
        pltpu.VMEM((5, B, F), jnp.float32),
        pltpu.VMEM_SHARED((N, F), jnp.float32),
        pltpu.SemaphoreType.DMA,
        pltpu.SemaphoreType.DMA,
        pltpu.SemaphoreType.DMA,
        pltpu.SemaphoreType.DMA,
        pltpu.SemaphoreType.DMA,
        pltpu.SemaphoreType.DMA,
        pltpu.SemaphoreType.DMA,
        pltpu.SemaphoreType.DMA,
        pltpu.SemaphoreType.DMA,
        pltpu.SemaphoreType.DMA,
        pltpu.SemaphoreType.DMA,
    ],
)
def _sc_agg_rows(y_hbm, src_hbm, dst_hbm, zeros_hbm, out_hbm,
                 src_v, dst_v, rbs, acc_sh,
                 semg0, semg1, semg2, semg3, semg4,
                 sems0, sems1, sems2, sems3, sems4, semz):
    cid = lax.axis_index("c")
    sid = lax.axis_index("s")
    wid = sid * NC + cid
    zcp = pltpu.async_copy(zeros_hbm, acc_sh.at[pl.ds(sid * RPT, RPT)], semz)
    pltpu.sync_copy(src_hbm.at[pl.ds(wid * EPW, EPW)], src_v)
    pltpu.sync_copy(dst_hbm.at[pl.ds(wid * EPW, EPW)], dst_v)
    zcp.wait()
    plsc.subcore_barrier()

    semg = (semg0, semg1, semg2, semg3, semg4)
    sems = (sems0, sems1, sems2, sems3, sems4)
    D = 5

    def start_gather(j, b):
        pltpu.async_copy(y_hbm.at[src_v.at[pl.ds(j * B, B)]], rbs.at[b],
                         semg[b])

    def start_scatter(j, b):
        pltpu.async_copy(rbs.at[b], acc_sh.at[dst_v.at[pl.ds(j * B, B)]],
                         sems[b], add=True)

    def wait_gather(b):
        pltpu.make_async_copy(y_hbm.at[src_v.at[pl.ds(0, B)]], rbs.at[b],
                              semg[b]).wait()

    def wait_scatter(b):
        pltpu.make_async_copy(rbs.at[b], acc_sh.at[dst_v.at[pl.ds(0, B)]],
                              sems[b]).wait()

    for b in range(D - 1):
        start_gather(b, b)
    wait_gather(0)
    start_scatter(0, 0)
    start_gather(D - 1, D - 1)

    def phase(j, b):
        wait_gather(b)
        start_scatter(j, b)
        wait_scatter((b + D - 1) % D)
        start_gather(j + D - 1, (b + D - 1) % D)

    def ring(t, c):
        j = 1 + D * t
        for p in range(D):
            phase(j + p, (1 + p) % D)
        return c

    Q = (NB - (2 * D - 2)) // D
    lax.fori_loop(0, Q, ring, 0)
    j0 = 1 + D * Q
    for p in range(NB - (D - 1) - j0):
        phase(j0 + p, (j0 + p) % D)
    for j in range(NB - (D - 1), NB):
        wait_gather(j % D)
        start_scatter(j, j % D)
    for j in range(NB - D, NB):
        wait_scatter(j % D)
    plsc.subcore_barrier()
    pltpu.sync_copy(acc_sh.at[pl.ds(sid * RPT, RPT)],
                    out_hbm.at[pl.ds(cid * N + sid * RPT, RPT)])


@functools.partial(
    pl.kernel,
    mesh=_mesh,
    compiler_params=_sc_params,
    out_type=jax.ShapeDtypeStruct((NW * N,), jnp.float32),
    scratch_types=[
        pltpu.VMEM((N,), jnp.float32),
        pltpu.VMEM((EPW,), jnp.int32),
        pltpu.VMEM((EPW,), jnp.int32),
        pltpu.VMEM((N,), jnp.float32),
        pltpu.VMEM((N,), jnp.float32),
        pltpu.VMEM((N,), jnp.float32),
        pltpu.VMEM((N,), jnp.float32),
    ],
)
def _sc_agg_scalar(q_hbm, src_hbm, dst_hbm, out_hbm, q_v, src_v, dst_v,
                   acc_v, acc_v1, acc_v2, acc_v3):
    wid = lax.axis_index("s") * NC + lax.axis_index("c")
    accs = (acc_v, acc_v1, acc_v2, acc_v3)
    pltpu.sync_copy(q_hbm, q_v)
    pltpu.sync_copy(src_hbm.at[pl.ds(wid * EPW, EPW)], src_v)
    pltpu.sync_copy(dst_hbm.at[pl.ds(wid * EPW, EPW)], dst_v)
    zero = jnp.zeros((16,), jnp.float32)

    def zbody(i, c):
        for a in accs:
            a[pl.ds(i * 16, 16)] = zero
        return c

    lax.fori_loop(0, N // 16, zbody, 0)

    def body(i, c):
        for u in range(4):
            s_ids = src_v[pl.ds((i * 4 + u) * 16, 16)]
            d_ids = dst_v[pl.ds((i * 4 + u) * 16, 16)]
            vals = plsc.load_gather(q_v, [s_ids])
            plsc.addupdate_scatter(accs[u], [d_ids], vals)
        return c

    lax.fori_loop(0, EPW // 64, body, 0)

    def mbody(i, c):
        sl = pl.ds(i * 16, 16)
        acc_v[sl] = ((acc_v[sl] + acc_v1[sl]) + (acc_v2[sl] + acc_v3[sl]))
        return c

    lax.fori_loop(0, N // 16, mbody, 0)
    pltpu.sync_copy(acc_v, out_hbm.at[pl.ds(wid * N, N)])


def _tc0_body(ei_ref, src_ref, dst_ref):
    src_ref[...] = ei_ref[0]
    dst_ref[...] = ei_ref[1]


def _tc1_body(x_ref, w_ref, deg_ref, y_ref, dis_ref):
    dsum = deg_ref[pl.ds(0, N)]
    for w in range(1, NW):
        dsum = dsum + deg_ref[pl.ds(w * N, N)]
    dis_row = lax.rsqrt(1.0 + dsum[None, :])
    dis_col = jnp.transpose(dis_row)
    xw = jnp.dot(x_ref[...], w_ref[...], preferred_element_type=jnp.float32)
    y_ref[...] = xw * dis_col
    dis_ref[...] = dis_row


def _tc2_body(agg_ref, y_ref, dis_ref, b1_ref, w2_ref, q_ref):
    agg = agg_ref[0] + agg_ref[1]
    dis_col = jnp.transpose(dis_ref[...])
    h = jnp.maximum(dis_col * (agg + y_ref[...]) + b1_ref[...], 0.0)
    q_col = jnp.dot(h, w2_ref[...],
                    preferred_element_type=jnp.float32) * dis_col
    q_ref[...] = jnp.transpose(q_col)[0]


def _tc3_body(a_ref, q_ref, dis_ref, b2_ref, o_ref):
    s = a_ref[pl.ds(0, N)]
    for w in range(1, NW):
        s = s + a_ref[pl.ds(w * N, N)]
    o_ref[...] = dis_ref[...][0] * (s + q_ref[...]) + b2_ref[0, 0]


def kernel(x, edge_index, W1, b1, W2, b2):
    ei = edge_index.astype(jnp.int32)

    src_row, dst_row = pl.pallas_call(
        _tc0_body,
        in_specs=[pl.BlockSpec((2, E), lambda: (0, 0))],
        out_specs=[
            pl.BlockSpec((E,), lambda: (0,)),
            pl.BlockSpec((E,), lambda: (0,)),
        ],
        out_shape=[
            jax.ShapeDtypeStruct((E,), jnp.int32),
            jax.ShapeDtypeStruct((E,), jnp.int32),
        ],
    )(ei)

    deg_parts = _sc_degree(dst_row)

    y, dis = pl.pallas_call(
        _tc1_body,
        in_specs=[
            pl.BlockSpec((N, F), lambda: (0, 0)),
            pl.BlockSpec((F, F), lambda: (0, 0)),
            pl.BlockSpec((NW * N,), lambda: (0,)),
        ],
        out_specs=[
            pl.BlockSpec((N, F), lambda: (0, 0)),
            pl.BlockSpec((1, N), lambda: (0, 0)),
        ],
        out_shape=[
            jax.ShapeDtypeStruct((N, F), jnp.float32),
            jax.ShapeDtypeStruct((1, N), jnp.float32),
        ],
    )(x, W1, deg_parts)

    zeros_tile = jnp.zeros((RPT, F), jnp.float32)
    agg1 = _sc_agg_rows(y, src_row, dst_row, zeros_tile)

    q = pl.pallas_call(
        _tc2_body,
        in_specs=[
            pl.BlockSpec((NC, N, F), lambda: (0, 0, 0)),
            pl.BlockSpec((N, F), lambda: (0, 0)),
            pl.BlockSpec((1, N), lambda: (0, 0)),
            pl.BlockSpec((1, F), lambda: (0, 0)),
            pl.BlockSpec((F, 1), lambda: (0, 0)),
        ],
        out_specs=pl.BlockSpec((N,), lambda: (0,)),
        out_shape=jax.ShapeDtypeStruct((N,), jnp.float32),
    )(agg1.reshape(NC, N, F), y, dis, b1.reshape(1, F), W2)

    agg2_parts = _sc_agg_scalar(q, src_row, dst_row)

    out = pl.pallas_call(
        _tc3_body,
        in_specs=[
            pl.BlockSpec((NW * N,), lambda: (0,)),
            pl.BlockSpec((N,), lambda: (0,)),
            pl.BlockSpec((1, N), lambda: (0, 0)),
            pl.BlockSpec((1, 1), lambda: (0, 0)),
        ],
        out_specs=pl.BlockSpec((N,), lambda: (0,)),
        out_shape=jax.ShapeDtypeStruct((N,), jnp.float32),
    )(agg2_parts, q, dis, b2.reshape(1, 1))

    return out


# speedup vs baseline: 1.0433x; 1.0433x over previous
import functools

import jax
import jax.numpy as jnp
from jax import lax
from jax.experimental import pallas as pl
from jax.experimental.pallas import tpu as pltpu
from jax.experimental.pallas import tpu_sc as plsc

N = 10000
F = 128
E = 320000
NC = 2
NS = 16
NW = NC * NS
EPW = E // NW
B = 40
NB = EPW // B
RPT = N // NS
RB = 1000

_mesh = plsc.VectorSubcoreMesh(core_axis_name="c", subcore_axis_name="s")
_sc_params = pltpu.CompilerParams(needs_layout_passes=False,
                                  use_tc_tiling_on_sc=False)


@functools.partial(
    pl.kernel,
    mesh=_mesh,
    compiler_params=_sc_params,
    out_type=jax.ShapeDtypeStruct((NW * N,), jnp.float32),
    scratch_types=[
        pltpu.VMEM((EPW,), jnp.int32),
        pltpu.VMEM((N,), jnp.float32),
    ],
)
def _sc_degree(dst_hbm, out_hbm, dst_v, acc_v):
    wid = lax.axis_index("s") * NC + lax.axis_index("c")
    pltpu.sync_copy(dst_hbm.at[pl.ds(wid * EPW, EPW)], dst_v)
    zero = jnp.zeros((16,), jnp.float32)
    one = jnp.ones((16,), jnp.float32)

    def zbody(i, c):
        acc_v[pl.ds(i * 16, 16)] = zero
        return c

    lax.fori_loop(0, N // 16, zbody, 0)

    def body(i, c):
        ids = dst_v[pl.ds(i * 16, 16)]
        plsc.addupdate_scatter(acc_v, [ids], one)
        return c

    lax.fori_loop(0, EPW // 16, body, 0)
    pltpu.sync_copy(acc_v, out_hbm.at[pl.ds(wid * N, N)])


@functools.partial(
    pl.kernel,
    mesh=_mesh,
    compiler_params=_sc_params,
    out_type=jax.ShapeDtypeStruct((NC * N, F), jnp.float32),
    scratch_types=[
        pltpu.VMEM((EPW,), jnp.int32),
        pltpu.VMEM((EPW,), jnp.int32),
        pltpu.VMEM((5, B, F), jnp.float32),
        pltpu.VMEM_SHARED((N, F), jnp.float32),
        pltpu.SemaphoreType.DMA,
        pltpu.SemaphoreType.DMA,
        pltpu.SemaphoreType.DMA,
        pltpu.SemaphoreType.DMA,
        pltpu.SemaphoreType.DMA,
        pltpu.SemaphoreType.DMA,
        pltpu.SemaphoreType.DMA,
        pltpu.SemaphoreType.DMA,
        pltpu.SemaphoreType.DMA,
        pltpu.SemaphoreType.DMA,
        pltpu.SemaphoreType.DMA,
    ],
)
def _sc_agg_rows(y_hbm, src_hbm, dst_hbm, zeros_hbm, out_hbm,
                 src_v, dst_v, rbs, acc_sh,
                 semg0, semg1, semg2, semg3, semg4,
                 sems0, sems1, sems2, sems3, sems4, semz):
    cid = lax.axis_index("c")
    sid = lax.axis_index("s")
    wid = sid * NC + cid
    zcp = pltpu.async_copy(zeros_hbm, acc_sh.at[pl.ds(sid * RPT, RPT)], semz)
    pltpu.sync_copy(src_hbm.at[pl.ds(wid * EPW, EPW)], src_v)
    pltpu.sync_copy(dst_hbm.at[pl.ds(wid * EPW, EPW)], dst_v)
    zcp.wait()
    plsc.subcore_barrier()

    semg = (semg0, semg1, semg2, semg3, semg4)
    sems = (sems0, sems1, sems2, sems3, sems4)
    D = 5

    def start_gather(j, b):
        pltpu.async_copy(y_hbm.at[src_v.at[pl.ds(j * B, B)]], rbs.at[b],
                         semg[b])

    def start_scatter(j, b):
        pltpu.async_copy(rbs.at[b], acc_sh.at[dst_v.at[pl.ds(j * B, B)]],
                         sems[b], add=True)

    def wait_gather(b):
        pltpu.make_async_copy(y_hbm.at[src_v.at[pl.ds(0, B)]], rbs.at[b],
                              semg[b]).wait()

    def wait_scatter(b):
        pltpu.make_async_copy(rbs.at[b], acc_sh.at[dst_v.at[pl.ds(0, B)]],
                              sems[b]).wait()

    for b in range(D - 1):
        start_gather(b, b)
    wait_gather(0)
    start_scatter(0, 0)
    start_gather(D - 1, D - 1)

    def phase(j, b):
        wait_gather(b)
        start_scatter(j, b)
        wait_scatter((b + D - 1) % D)
        start_gather(j + D - 1, (b + D - 1) % D)

    def ring(t, c):
        j = 1 + D * t
        for p in range(D):
            phase(j + p, (1 + p) % D)
        return c

    Q = (NB - (2 * D - 2)) // D
    lax.fori_loop(0, Q, ring, 0)
    j0 = 1 + D * Q
    for p in range(NB - (D - 1) - j0):
        phase(j0 + p, (j0 + p) % D)
    for j in range(NB - (D - 1), NB):
        wait_gather(j % D)
        start_scatter(j, j % D)
    for j in range(NB - D, NB):
        wait_scatter(j % D)
    plsc.subcore_barrier()
    pltpu.sync_copy(acc_sh.at[pl.ds(sid * RPT, RPT)],
                    out_hbm.at[pl.ds(cid * N + sid * RPT, RPT)])


@functools.partial(
    pl.kernel,
    mesh=_mesh,
    compiler_params=_sc_params,
    out_type=jax.ShapeDtypeStruct((NW * N,), jnp.float32),
    scratch_types=[
        pltpu.VMEM((N,), jnp.float32),
        pltpu.VMEM((EPW,), jnp.int32),
        pltpu.VMEM((EPW,), jnp.int32),
        pltpu.VMEM((N,), jnp.float32),
    ],
)
def _sc_agg_scalar(q_hbm, src_hbm, dst_hbm, out_hbm, q_v, src_v, dst_v, acc_v):
    wid = lax.axis_index("s") * NC + lax.axis_index("c")
    pltpu.sync_copy(q_hbm, q_v)
    pltpu.sync_copy(src_hbm.at[pl.ds(wid * EPW, EPW)], src_v)
    pltpu.sync_copy(dst_hbm.at[pl.ds(wid * EPW, EPW)], dst_v)
    zero = jnp.zeros((16,), jnp.float32)

    def zbody(i, c):
        acc_v[pl.ds(i * 16, 16)] = zero
        return c

    lax.fori_loop(0, N // 16, zbody, 0)

    def body(i, c):
        s_ids = src_v[pl.ds(i * 16, 16)]
        d_ids = dst_v[pl.ds(i * 16, 16)]
        vals = plsc.load_gather(q_v, [s_ids])
        plsc.addupdate_scatter(acc_v, [d_ids], vals)
        return c

    lax.fori_loop(0, EPW // 16, body, 0)
    pltpu.sync_copy(acc_v, out_hbm.at[pl.ds(wid * N, N)])


def _tc0_body(ei_ref, src_ref, dst_ref):
    src_ref[...] = ei_ref[0]
    dst_ref[...] = ei_ref[1]


def _tc1_body(x_ref, w_ref, deg_ref, y_ref, dis_ref):
    dsum = deg_ref[pl.ds(0, N)]
    for w in range(1, NW):
        dsum = dsum + deg_ref[pl.ds(w * N, N)]
    dis_row = lax.rsqrt(1.0 + dsum[None, :])
    dis_col = jnp.transpose(dis_row)
    xw = jnp.dot(x_ref[...], w_ref[...], preferred_element_type=jnp.float32)
    y_ref[...] = xw * dis_col
    dis_ref[...] = dis_row


def _tc2_body(agg_ref, y_ref, dis_ref, b1_ref, w2_ref, q_ref):
    agg = agg_ref[0] + agg_ref[1]
    dis_col = jnp.transpose(dis_ref[...])
    h = jnp.maximum(dis_col * (agg + y_ref[...]) + b1_ref[...], 0.0)
    q_col = jnp.dot(h, w2_ref[...],
                    preferred_element_type=jnp.float32) * dis_col
    q_ref[...] = jnp.transpose(q_col)[0]


def _tc3_body(a_ref, q_ref, dis_ref, b2_ref, o_ref):
    s = a_ref[pl.ds(0, N)]
    for w in range(1, NW):
        s = s + a_ref[pl.ds(w * N, N)]
    o_ref[...] = dis_ref[...][0] * (s + q_ref[...]) + b2_ref[0, 0]


def kernel(x, edge_index, W1, b1, W2, b2):
    ei = edge_index.astype(jnp.int32)

    src_row, dst_row = pl.pallas_call(
        _tc0_body,
        in_specs=[pl.BlockSpec((2, E), lambda: (0, 0))],
        out_specs=[
            pl.BlockSpec((E,), lambda: (0,)),
            pl.BlockSpec((E,), lambda: (0,)),
        ],
        out_shape=[
            jax.ShapeDtypeStruct((E,), jnp.int32),
            jax.ShapeDtypeStruct((E,), jnp.int32),
        ],
    )(ei)

    deg_parts = _sc_degree(dst_row)

    y, dis = pl.pallas_call(
        _tc1_body,
        in_specs=[
            pl.BlockSpec((N, F), lambda: (0, 0)),
            pl.BlockSpec((F, F), lambda: (0, 0)),
            pl.BlockSpec((NW * N,), lambda: (0,)),
        ],
        out_specs=[
            pl.BlockSpec((N, F), lambda: (0, 0)),
            pl.BlockSpec((1, N), lambda: (0, 0)),
        ],
        out_shape=[
            jax.ShapeDtypeStruct((N, F), jnp.float32),
            jax.ShapeDtypeStruct((1, N), jnp.float32),
        ],
    )(x, W1, deg_parts)

    zeros_tile = jnp.zeros((RPT, F), jnp.float32)
    agg1 = _sc_agg_rows(y, src_row, dst_row, zeros_tile)

    q = pl.pallas_call(
        _tc2_body,
        in_specs=[
            pl.BlockSpec((NC, N, F), lambda: (0, 0, 0)),
            pl.BlockSpec((N, F), lambda: (0, 0)),
            pl.BlockSpec((1, N), lambda: (0, 0)),
            pl.BlockSpec((1, F), lambda: (0, 0)),
            pl.BlockSpec((F, 1), lambda: (0, 0)),
        ],
        out_specs=pl.BlockSpec((N,), lambda: (0,)),
        out_shape=jax.ShapeDtypeStruct((N,), jnp.float32),
    )(agg1.reshape(NC, N, F), y, dis, b1.reshape(1, F), W2)

    agg2_parts = _sc_agg_scalar(q, src_row, dst_row)

    out = pl.pallas_call(
        _tc3_body,
        in_specs=[
            pl.BlockSpec((NW * N,), lambda: (0,)),
            pl.BlockSpec((N,), lambda: (0,)),
            pl.BlockSpec((1, N), lambda: (0, 0)),
            pl.BlockSpec((1, 1), lambda: (0, 0)),
        ],
        out_specs=pl.BlockSpec((N,), lambda: (0,)),
        out_shape=jax.ShapeDtypeStruct((N,), jnp.float32),
    )(agg2_parts, q, dis, b2.reshape(1, 1))

    return out
